# baseline (speedup 1.0000x reference)
import functools

import jax
import jax.numpy as jnp
from jax.experimental import pallas as pl
from jax.experimental.pallas import tpu as pltpu

_N0, _N1 = 64, 64
_M0, _M1 = 64, 64
_R = 16
_BN = 256


def _permmm_body(x_ref, p_ref, b_ref, o_ref):
    f_in = x_ref.shape[1]
    rows = jax.lax.broadcasted_iota(jnp.int32, (f_in, p_ref.shape[1]), 0)
    oh = jnp.where(rows == p_ref[...], 1.0, 0.0).astype(jnp.bfloat16)
    acc = jnp.dot(
        x_ref[...].astype(jnp.bfloat16), oh, preferred_element_type=jnp.float32
    )
    o_ref[...] = (acc + b_ref[...]).astype(o_ref.dtype)


@functools.partial(jax.jit, static_argnames=("out_dtype",))
def _perm_matmul(x, perm2d, bias2d, out_dtype):
    b, f_in = x.shape
    f_out = perm2d.shape[1]
    return pl.pallas_call(
        _permmm_body,
        grid=(f_out // _BN,),
        in_specs=[
            pl.BlockSpec((b, f_in), lambda n: (0, 0)),
            pl.BlockSpec((1, _BN), lambda n: (0, n)),
            pl.BlockSpec((1, _BN), lambda n: (0, n)),
        ],
        out_specs=pl.BlockSpec((b, _BN), lambda n: (0, n)),
        out_shape=jax.ShapeDtypeStruct((b, f_out), out_dtype),
        compiler_params=pltpu.CompilerParams(
            dimension_semantics=("parallel",),
        ),
    )(x, perm2d, bias2d)


def _mm_body(x_ref, v_ref, o_ref):
    o_ref[...] = jnp.dot(
        x_ref[...], v_ref[...], preferred_element_type=jnp.float32
    ).astype(o_ref.dtype)


@jax.jit
def _matmul(x, v):
    b, f_in = x.shape
    f_out = v.shape[1]
    return pl.pallas_call(
        _mm_body,
        grid=(f_out // _BN,),
        in_specs=[
            pl.BlockSpec((b, f_in), lambda n: (0, 0)),
            pl.BlockSpec((f_in, _BN), lambda n: (0, n)),
        ],
        out_specs=pl.BlockSpec((b, _BN), lambda n: (0, n)),
        out_shape=jax.ShapeDtypeStruct((b, f_out), jnp.bfloat16),
        compiler_params=pltpu.CompilerParams(
            dimension_semantics=("parallel",),
        ),
    )(x, v)


def kernel(x, g0, g1, alpha, per_dim_scale, bias, input_perm, output_inv_perm):
    s4 = (alpha * per_dim_scale).reshape(_M0, _M1)
    v4 = jnp.einsum("ijr,rmk->jkim", g0[0], g1[..., 0]) * s4[None, None]
    v = v4.reshape(_N0 * _N1, _M0 * _M1).astype(jnp.bfloat16)
    zeros = jnp.zeros((1, x.shape[1]), jnp.float32)
    xp = _perm_matmul(x, input_perm.reshape(1, -1), zeros, jnp.bfloat16)
    y_pre = _matmul(xp, v)
    return _perm_matmul(
        y_pre, output_inv_perm.reshape(1, -1), bias.reshape(1, -1), jnp.float32
    )


# speedup vs baseline: 1.0628x; 1.0628x over previous
import functools

import jax
import jax.numpy as jnp
from jax.experimental import pallas as pl
from jax.experimental.pallas import tpu as pltpu

_N0, _N1 = 64, 64
_M0, _M1 = 64, 64
_R = 16
_BN = 256


def _permmm_body(x_ref, p_ref, b_ref, o_ref):
    f_in = x_ref.shape[1]
    rows = jax.lax.broadcasted_iota(jnp.int32, (f_in, p_ref.shape[1]), 0)
    oh = jnp.where(rows == p_ref[...], 1.0, 0.0).astype(jnp.bfloat16)
    acc = jnp.dot(
        x_ref[...].astype(jnp.bfloat16), oh, preferred_element_type=jnp.float32
    )
    o_ref[...] = (acc + b_ref[...]).astype(o_ref.dtype)


@functools.partial(jax.jit, static_argnames=("out_dtype",))
def _perm_matmul(x, perm2d, bias2d, out_dtype):
    b, f_in = x.shape
    f_out = perm2d.shape[1]
    return pl.pallas_call(
        _permmm_body,
        grid=(f_out // _BN,),
        in_specs=[
            pl.BlockSpec((b, f_in), lambda n: (0, 0)),
            pl.BlockSpec((1, _BN), lambda n: (0, n)),
            pl.BlockSpec((1, _BN), lambda n: (0, n)),
        ],
        out_specs=pl.BlockSpec((b, _BN), lambda n: (0, n)),
        out_shape=jax.ShapeDtypeStruct((b, f_out), out_dtype),
        compiler_params=pltpu.CompilerParams(
            dimension_semantics=("parallel",),
        ),
    )(x, perm2d, bias2d)


def _mm_body(x_ref, v_ref, o_ref):
    o_ref[...] = jnp.dot(
        x_ref[...], v_ref[...], preferred_element_type=jnp.float32
    ).astype(o_ref.dtype)


@jax.jit
def _matmul(x, v):
    b, f_in = x.shape
    f_out = v.shape[1]
    return pl.pallas_call(
        _mm_body,
        grid=(f_out // _BN,),
        in_specs=[
            pl.BlockSpec((b, f_in), lambda n: (0, 0)),
            pl.BlockSpec((f_in, _BN), lambda n: (0, n)),
        ],
        out_specs=pl.BlockSpec((b, _BN), lambda n: (0, n)),
        out_shape=jax.ShapeDtypeStruct((b, f_out), jnp.bfloat16),
        compiler_params=pltpu.CompilerParams(
            dimension_semantics=("parallel",),
        ),
    )(x, v)


def kernel(x, g0, g1, alpha, per_dim_scale, bias, input_perm, output_inv_perm):
    s4 = (alpha * per_dim_scale).reshape(_M0, _M1)
    t1 = jnp.einsum("ijr,rmk->ijmk", g0[0], g1[..., 0]) * s4[:, None, :, None]
    v = t1.astype(jnp.bfloat16).transpose(1, 3, 0, 2).reshape(
        _N0 * _N1, _M0 * _M1
    )
    zeros = jnp.zeros((1, x.shape[1]), jnp.float32)
    xp = _perm_matmul(x, input_perm.reshape(1, -1), zeros, jnp.bfloat16)
    y_pre = _matmul(xp, v)
    return _perm_matmul(
        y_pre, output_inv_perm.reshape(1, -1), bias.reshape(1, -1), jnp.float32
    )
